# baseline (speedup 1.0000x reference)
import jax
import jax.numpy as jnp
from jax.experimental import pallas as pl
from jax.experimental.pallas import tpu as pltpu

K = 16
D = 128


_NCHUNK = 16
_CD = D // _NCHUNK


def _select_body(conf_ref, dens_ref, out_ref, sem):
    i = conf_ref[0]
    copies = []
    for c in range(_NCHUNK):
        sl = pl.ds(c * _CD, _CD)
        copies.append(
            pltpu.make_async_copy(dens_ref.at[i, sl], out_ref.at[sl], sem.at[c])
        )
    for cp in copies:
        cp.start()
    for cp in copies:
        cp.wait()


def kernel(density, conformation):
    conf = jnp.atleast_1d(jnp.asarray(conformation, jnp.int32))
    grid_spec = pltpu.PrefetchScalarGridSpec(
        num_scalar_prefetch=1,
        grid=(1,),
        in_specs=[pl.BlockSpec(memory_space=pl.ANY)],
        out_specs=pl.BlockSpec(memory_space=pl.ANY),
        scratch_shapes=[pltpu.SemaphoreType.DMA((_NCHUNK,))],
    )
    return pl.pallas_call(
        _select_body,
        grid_spec=grid_spec,
        out_shape=jax.ShapeDtypeStruct((D, D, D), density.dtype),
    )(conf, density)


# speedup vs baseline: 26.4862x; 26.4862x over previous
import jax
import jax.numpy as jnp
from jax.experimental import pallas as pl
from jax.experimental.pallas import tpu as pltpu

K = 16
D = 128


_BD = 16


def _select_body(conf_ref, dens_ref, out_ref):
    out_ref[...] = dens_ref[0]


def kernel(density, conformation):
    conf = jnp.atleast_1d(jnp.asarray(conformation, jnp.int32))
    grid_spec = pltpu.PrefetchScalarGridSpec(
        num_scalar_prefetch=1,
        grid=(D // _BD,),
        in_specs=[
            pl.BlockSpec((1, _BD, D, D), lambda g, conf: (conf[0], g, 0, 0))
        ],
        out_specs=pl.BlockSpec((_BD, D, D), lambda g, conf: (g, 0, 0)),
    )
    return pl.pallas_call(
        _select_body,
        grid_spec=grid_spec,
        out_shape=jax.ShapeDtypeStruct((D, D, D), density.dtype),
    )(conf, density)


# speedup vs baseline: 31.6296x; 1.1942x over previous
import jax
import jax.numpy as jnp
from jax.experimental import pallas as pl
from jax.experimental.pallas import tpu as pltpu

K = 16
D = 128


_BD = 32


def _select_body(conf_ref, dens_ref, out_ref):
    out_ref[...] = dens_ref[0]


def kernel(density, conformation):
    conf = jnp.atleast_1d(jnp.asarray(conformation, jnp.int32))
    grid_spec = pltpu.PrefetchScalarGridSpec(
        num_scalar_prefetch=1,
        grid=(D // _BD,),
        in_specs=[
            pl.BlockSpec((1, _BD, D, D), lambda g, conf: (conf[0], g, 0, 0))
        ],
        out_specs=pl.BlockSpec((_BD, D, D), lambda g, conf: (g, 0, 0)),
    )
    return pl.pallas_call(
        _select_body,
        grid_spec=grid_spec,
        out_shape=jax.ShapeDtypeStruct((D, D, D), density.dtype),
    )(conf, density)


# speedup vs baseline: 39.1398x; 1.2374x over previous
import jax
import jax.numpy as jnp
from jax.experimental import pallas as pl
from jax.experimental.pallas import tpu as pltpu

K = 16
D = 128


_BD = 64


def _select_body(conf_ref, dens_ref, out_ref):
    out_ref[...] = dens_ref[0]


def kernel(density, conformation):
    conf = jnp.atleast_1d(jnp.asarray(conformation, jnp.int32))
    grid_spec = pltpu.PrefetchScalarGridSpec(
        num_scalar_prefetch=1,
        grid=(D // _BD,),
        in_specs=[
            pl.BlockSpec((1, _BD, D, D), lambda g, conf: (conf[0], g, 0, 0))
        ],
        out_specs=pl.BlockSpec((_BD, D, D), lambda g, conf: (g, 0, 0)),
    )
    return pl.pallas_call(
        _select_body,
        grid_spec=grid_spec,
        out_shape=jax.ShapeDtypeStruct((D, D, D), density.dtype),
    )(conf, density)
